# baseline (speedup 1.0000x reference)
import functools

import jax
import jax.numpy as jnp
from jax import lax
from jax.experimental import pallas as pl
from jax.experimental.pallas import tpu as pltpu
import jax.experimental.pallas.tpu_sc as plsc

_Q, _N, _F = 1024, 20000, 64
_NPAD = 20480
_NB = 2048
_NBLK = _NPAD // _NB
_BIGF = 3.0e38


def _argmin_body(a_ref, p_ref, a2_ref, b2_ref, idx_out, min_ref, idx_ref):
    nb = pl.program_id(1)
    a = a_ref[0]
    p = p_ref[0]
    mm = lax.dot_general(a, p, (((1,), (1,)), ((), ())),
                         preferred_element_type=jnp.float32)
    d = a2_ref[0] + b2_ref[0] - 2.0 * mm
    col = lax.broadcasted_iota(jnp.int32, (1, _NB), 1) + nb * _NB
    d = jnp.where(col < _N, d, _BIGF)
    bmin = jnp.min(d, axis=1, keepdims=True)
    bidx = jnp.min(
        jnp.where(d == bmin, jnp.broadcast_to(col, d.shape), jnp.int32(2**30)),
        axis=1, keepdims=True)

    @pl.when(nb == 0)
    def _():
        min_ref[...] = jnp.full_like(min_ref, _BIGF)
        idx_ref[...] = jnp.zeros_like(idx_ref)

    better = bmin < min_ref[...]
    min_ref[...] = jnp.where(better, bmin, min_ref[...])
    idx_ref[...] = jnp.where(better, bidx, idx_ref[...])

    @pl.when(nb == _NBLK - 1)
    def _():
        idx_out[0] = idx_ref[...]


def _argmin_call(A, P, A2, B2):
    return pl.pallas_call(
        _argmin_body,
        grid=(2, _NBLK),
        in_specs=[
            pl.BlockSpec((1, _Q, 3), lambda s, nb: (s, 0, 0)),
            pl.BlockSpec((1, _NB, 3), lambda s, nb: (s, nb, 0)),
            pl.BlockSpec((1, _Q, 1), lambda s, nb: (s, 0, 0)),
            pl.BlockSpec((1, 1, _NB), lambda s, nb: (s, 0, nb)),
        ],
        out_specs=pl.BlockSpec((1, _Q, 1), lambda s, nb: (s, 0, 0)),
        out_shape=jax.ShapeDtypeStruct((2, _Q, 1), jnp.int32),
        scratch_shapes=[pltpu.VMEM((_Q, 1), jnp.float32),
                        pltpu.VMEM((_Q, 1), jnp.int32)],
        compiler_params=pltpu.CompilerParams(
            dimension_semantics=("arbitrary", "arbitrary")),
    )(A, P, A2, B2)


_SC_NC, _SC_NS = 2, 16
_NW = _SC_NC * _SC_NS
_BPW = _Q // _NW


def _sc_gather(feats_l, feats_r, idx_l, idx_r):
    mesh = plsc.VectorSubcoreMesh(core_axis_name="c", subcore_axis_name="s",
                                  num_cores=_SC_NC, num_subcores=_SC_NS)

    @functools.partial(
        pl.kernel,
        out_type=[jax.ShapeDtypeStruct((_Q, _F), jnp.float32),
                  jax.ShapeDtypeStruct((_Q, _F), jnp.float32)],
        mesh=mesh,
        scratch_types=[pltpu.VMEM((_BPW,), jnp.int32),
                       pltpu.VMEM((_BPW, _F), jnp.float32),
                       pltpu.VMEM((_BPW,), jnp.int32),
                       pltpu.VMEM((_BPW, _F), jnp.float32),
                       pltpu.SemaphoreType.DMA,
                       pltpu.SemaphoreType.DMA],
        compiler_params=pltpu.CompilerParams(use_tc_tiling_on_sc=False),
    )
    def k(fl_hbm, fr_hbm, il_hbm, ir_hbm, ol_hbm, or_hbm,
          il_v, rl_v, ir_v, rr_v, sem_l, sem_r):
        wid = lax.axis_index("s") * _SC_NC + lax.axis_index("c")
        base = wid * _BPW
        pltpu.sync_copy(il_hbm.at[pl.ds(base, _BPW)], il_v)
        pltpu.sync_copy(ir_hbm.at[pl.ds(base, _BPW)], ir_v)
        cl = pltpu.async_copy(fl_hbm.at[il_v], rl_v, sem_l)
        cr = pltpu.async_copy(fr_hbm.at[ir_v], rr_v, sem_r)
        cl.wait()
        cr.wait()
        pltpu.sync_copy(rl_v, ol_hbm.at[pl.ds(base, _BPW)])
        pltpu.sync_copy(rr_v, or_hbm.at[pl.ds(base, _BPW)])

    return k(feats_l, feats_r, idx_l, idx_r)


def _mlp_body(gl_ref, gr_ref, w1a_ref, w1b_ref, b1_ref, w2_ref, b2_ref, o_ref):
    h = (jnp.dot(gl_ref[...], w1a_ref[...], preferred_element_type=jnp.float32)
         + jnp.dot(gr_ref[...], w1b_ref[...], preferred_element_type=jnp.float32)
         + b1_ref[...])
    h = jnp.maximum(h, 0.0)
    o_ref[...] = (jnp.dot(h, w2_ref[...], preferred_element_type=jnp.float32)
                  + b2_ref[...])


def _mlp_call(gl, gr, w1a, w1b, b1, w2, b2):
    return pl.pallas_call(
        _mlp_body,
        out_shape=jax.ShapeDtypeStruct((_Q, 1), jnp.float32),
    )(gl, gr, w1a, w1b, b1, w2, b2)


def kernel(locs_left, locs_right, pos_left, pos_right,
           feats_left, feats_right, W1, b1, W2, b2):
    A = jnp.stack([locs_left, locs_right])
    P = jnp.stack([
        jnp.pad(pos_left, ((0, _NPAD - _N), (0, 0))),
        jnp.pad(pos_right, ((0, _NPAD - _N), (0, 0))),
    ])
    a2l = jnp.sum(locs_left * locs_left, axis=1, keepdims=True)
    a2r = jnp.sum(locs_right * locs_right, axis=1, keepdims=True)
    A2 = jnp.stack([a2l, a2r])
    b2l = jnp.sum(pos_left * pos_left, axis=1)
    b2r = jnp.sum(pos_right * pos_right, axis=1)
    B2 = jnp.stack([jnp.pad(b2l, (0, _NPAD - _N)),
                    jnp.pad(b2r, (0, _NPAD - _N))]).reshape(2, 1, _NPAD)

    idx3 = _argmin_call(A, P, A2, B2)
    gl, gr = _sc_gather(feats_left, feats_right,
                        idx3[0, :, 0], idx3[1, :, 0])
    out = _mlp_call(gl, gr, W1[:_F], W1[_F:], b1.reshape(1, -1),
                    W2, b2.reshape(1, 1))
    return out.reshape(-1)
